# baseline (speedup 1.0000x reference)
import functools

import jax
import jax.numpy as jnp
from jax import lax
from jax.experimental import pallas as pl
from jax.experimental.pallas import tpu as pltpu
from jax.experimental.pallas import tpu_sc as plsc

B = 4096
NTOK = 26
NFLT = 13
SEQ = 200
VOCAB = 100000

TTOT = NTOK * VOCAB
TCUT = (TTOT // 128) * 128
TTAIL = TTOT - TCUT
SCUT = (VOCAB // 128) * 128
STAIL = VOCAB - SCUT

NC = 2
NS = 16
NW = NC * NS
RPW = B // NW
L = 16
G = RPW // L
CH = 25
NCHUNK = SEQ // CH

_mesh = plsc.VectorSubcoreMesh(core_axis_name="c", subcore_axis_name="s")


@functools.partial(
    pl.kernel,
    out_type=jax.ShapeDtypeStruct((B,), jnp.float32),
    mesh=_mesh,
    compiler_params=pltpu.CompilerParams(needs_layout_passes=False),
    scratch_types=[
        pltpu.VMEM((NTOK * RPW,), jnp.int32),
        pltpu.VMEM((NTOK * RPW,), jnp.int32),
        pltpu.VMEM((NTOK * RPW,), jnp.float32),
        pltpu.VMEM((VOCAB,), jnp.float32),
        pltpu.VMEM((CH * RPW,), jnp.int32),
        pltpu.VMEM((CH * RPW,), jnp.int32),
        pltpu.VMEM((NFLT * RPW,), jnp.float32),
        pltpu.VMEM((1, 128), jnp.float32),
        pltpu.VMEM((RPW,), jnp.float32),
        pltpu.SemaphoreType.DMA,
        pltpu.SemaphoreType.DMA,
        pltpu.SemaphoreType.DMA,
        pltpu.SemaphoreType.DMA,
        pltpu.SemaphoreType.DMA,
    ],
)
def _fm_first_order(tok_idx_cl_hbm, tok_idx_hbm, seq_idx_hbm, flt_hbm,
                    ttab_hbm, stab_hbm, aux_hbm, out_hbm,
                    idx_cl_v, idx_tok_v, val_tok_v, stab_v, ib0, ib1,
                    flt_v, aux_v, out_v,
                    sem_tok, sem_tab, sem_tab2, sem_a, sem_b):
    lane = lax.iota(jnp.int32, L)
    zero_i = jnp.zeros((L,), jnp.int32)
    wid = lax.axis_index("s") * NC + lax.axis_index("c")
    base = wid * RPW
    sbase = wid * SEQ * RPW

    half = SCUT // 2
    cp_tab = pltpu.async_copy(stab_hbm.at[pl.ds(0, half)],
                              stab_v.at[pl.ds(0, half)], sem_tab)
    cp_tab2 = pltpu.async_copy(stab_hbm.at[pl.ds(half, SCUT - half)],
                               stab_v.at[pl.ds(half, SCUT - half)], sem_tab2)
    pltpu.sync_copy(tok_idx_cl_hbm.at[pl.ds(wid * NTOK * RPW, NTOK * RPW)],
                    idx_cl_v)
    cp_tok = pltpu.async_copy(ttab_hbm.at[idx_cl_v], val_tok_v, sem_tok)
    ibufs = [ib0, ib1]
    sems = [sem_a, sem_b]
    cps = [pltpu.async_copy(seq_idx_hbm.at[pl.ds(sbase + c * CH * RPW,
                                                 CH * RPW)],
                            ibufs[c], sems[c])
           for c in range(2)]
    pltpu.sync_copy(tok_idx_hbm.at[pl.ds(wid * NTOK * RPW, NTOK * RPW)],
                    idx_tok_v)
    pltpu.sync_copy(flt_hbm.at[pl.ds(wid * NFLT * RPW, NFLT * RPW)], flt_v)
    pltpu.sync_copy(aux_hbm, aux_v)
    for i in range(STAIL // L):
        stab_v[pl.ds(SCUT + i * L, L)] = plsc.load_gather(
            aux_v, [zero_i, lane + (TTAIL + i * L)])

    wj = [plsc.load_gather(aux_v, [zero_i, jnp.full((L,), TTAIL + STAIL + j,
                                                    jnp.int32)])
          for j in range(NFLT + 1)]
    accs = []
    for g in range(G):
        acc = wj[NFLT]
        for j in range(NFLT):
            acc = acc + wj[j] * flt_v[pl.ds(j * RPW + g * L, L)]
        accs.append(acc)

    cp_tok.wait()
    for g in range(G):
        acc = accs[g]
        for j in range(NTOK):
            sl = pl.ds(j * RPW + g * L, L)
            ids = idx_tok_v[sl]
            av = val_tok_v[sl]
            tix = jnp.maximum(ids - TCUT, 0)
            tv = plsc.load_gather(aux_v, [zero_i, tix])
            acc = acc + jnp.where(ids >= TCUT, tv, av)
        accs[g] = acc

    cp_tab.wait()
    cp_tab2.wait()
    zero = jnp.zeros((L,), jnp.float32)

    for c in range(NCHUNK):
        cps[c % 2].wait()
        buf = ibufs[c % 2]

        def body(j, accs, buf=buf):
            out = []
            for g in range(G):
                ids = buf[pl.ds(j * RPW + g * L, L)]
                vs = plsc.load_gather(stab_v, [ids])
                out.append(accs[g] + jnp.where(ids != 0, vs, zero))
            return tuple(out)

        accs = list(lax.fori_loop(0, CH, body, tuple(accs)))
        if c + 2 < NCHUNK:
            cps[c % 2] = pltpu.async_copy(
                seq_idx_hbm.at[pl.ds(sbase + (c + 2) * CH * RPW, CH * RPW)],
                ibufs[c % 2], sems[c % 2])

    for g in range(G):
        out_v[pl.ds(g * L, L)] = accs[g]
    pltpu.sync_copy(out_v, out_hbm.at[pl.ds(base, RPW)])


def kernel(float_fields, token_fields, token_seq, token_table, float_table,
           seq_table, bias):
    def permute(x):
        return x.reshape(NW, RPW, -1).transpose(0, 2, 1).reshape(-1)

    offsets = (jnp.arange(NTOK, dtype=jnp.int32) * VOCAB)[None, :]
    tok_idx = permute(token_fields.astype(jnp.int32) + offsets)
    tok_idx_cl = jnp.minimum(tok_idx, TCUT - 1)
    seq_idx = permute(token_seq.astype(jnp.int32))
    flt_t = permute(float_fields)

    ttab_a = lax.slice(token_table, (0, 0), (TCUT, 1)).reshape(-1)
    stab_a = lax.slice(seq_table, (0, 0), (SCUT, 1)).reshape(-1)
    aux = jnp.concatenate([
        lax.slice(token_table, (TCUT, 0), (TTOT, 1)).T,
        lax.slice(seq_table, (SCUT, 0), (VOCAB, 1)).T,
        float_table.T,
        bias[None, :],
        jnp.zeros((1, 128 - TTAIL - STAIL - NFLT - 1), jnp.float32)],
        axis=1)

    out = _fm_first_order(tok_idx_cl, tok_idx, seq_idx, flt_t,
                          ttab_a, stab_a, aux)
    return out.reshape(B, 1)


# speedup vs baseline: 1.0045x; 1.0045x over previous
import functools

import jax
import jax.numpy as jnp
from jax import lax
from jax.experimental import pallas as pl
from jax.experimental.pallas import tpu as pltpu
from jax.experimental.pallas import tpu_sc as plsc

B = 4096
NTOK = 26
NFLT = 13
SEQ = 200
VOCAB = 100000

TTOT = NTOK * VOCAB
TCUT = (TTOT // 128) * 128
TTAIL = TTOT - TCUT
SCUT = (VOCAB // 128) * 128
STAIL = VOCAB - SCUT

NC = 2
NS = 16
NW = NC * NS
RPW = B // NW
L = 16
G = RPW // L
CH = 25
NCHUNK = SEQ // CH

_mesh = plsc.VectorSubcoreMesh(core_axis_name="c", subcore_axis_name="s")


@functools.partial(
    pl.kernel,
    out_type=jax.ShapeDtypeStruct((B,), jnp.float32),
    mesh=_mesh,
    compiler_params=pltpu.CompilerParams(needs_layout_passes=False),
    scratch_types=[
        pltpu.VMEM((NTOK * RPW,), jnp.int32),
        pltpu.VMEM((NTOK * RPW,), jnp.int32),
        pltpu.VMEM((NTOK * RPW,), jnp.float32),
        pltpu.VMEM((VOCAB,), jnp.float32),
        pltpu.VMEM((CH * RPW,), jnp.int32),
        pltpu.VMEM((CH * RPW,), jnp.int32),
        pltpu.VMEM((NFLT * RPW,), jnp.float32),
        pltpu.VMEM((1, 128), jnp.float32),
        pltpu.VMEM((RPW,), jnp.float32),
        pltpu.SemaphoreType.DMA,
        pltpu.SemaphoreType.DMA,
        pltpu.SemaphoreType.DMA,
        pltpu.SemaphoreType.DMA,
        pltpu.SemaphoreType.DMA,
    ],
)
def _fm_first_order(tok_idx_cl_hbm, tok_idx_hbm, seq_idx_hbm, flt_hbm,
                    ttab_hbm, stab_hbm, aux_hbm, out_hbm,
                    idx_cl_v, idx_tok_v, val_tok_v, stab_v, ib0, ib1,
                    flt_v, aux_v, out_v,
                    sem_tok, sem_tab, sem_tab2, sem_a, sem_b):
    lane = lax.iota(jnp.int32, L)
    zero_i = jnp.zeros((L,), jnp.int32)
    wid = lax.axis_index("s") * NC + lax.axis_index("c")
    base = wid * RPW
    sbase = wid * SEQ * RPW

    half = SCUT // 2
    cp_tab = pltpu.async_copy(stab_hbm.at[pl.ds(0, half)],
                              stab_v.at[pl.ds(0, half)], sem_tab)
    cp_tab2 = pltpu.async_copy(stab_hbm.at[pl.ds(half, SCUT - half)],
                               stab_v.at[pl.ds(half, SCUT - half)], sem_tab2)
    pltpu.sync_copy(tok_idx_cl_hbm.at[pl.ds(wid * NTOK * RPW, NTOK * RPW)],
                    idx_cl_v)
    cp_tok = pltpu.async_copy(ttab_hbm.at[idx_cl_v], val_tok_v, sem_tok)
    ibufs = [ib0, ib1]
    sems = [sem_a, sem_b]
    cps = [pltpu.async_copy(seq_idx_hbm.at[pl.ds(sbase + c * CH * RPW,
                                                 CH * RPW)],
                            ibufs[c], sems[c])
           for c in range(2)]
    pltpu.sync_copy(tok_idx_hbm.at[pl.ds(wid * NTOK * RPW, NTOK * RPW)],
                    idx_tok_v)
    pltpu.sync_copy(flt_hbm.at[pl.ds(wid * NFLT * RPW, NFLT * RPW)], flt_v)
    pltpu.sync_copy(aux_hbm, aux_v)
    for i in range(STAIL // L):
        stab_v[pl.ds(SCUT + i * L, L)] = plsc.load_gather(
            aux_v, [zero_i, lane + (TTAIL + i * L)])

    wj = [plsc.load_gather(aux_v, [zero_i, jnp.full((L,), TTAIL + STAIL + j,
                                                    jnp.int32)])
          for j in range(NFLT + 1)]
    accs = []
    for g in range(G):
        acc = wj[NFLT]
        for j in range(NFLT):
            acc = acc + wj[j] * flt_v[pl.ds(j * RPW + g * L, L)]
        accs.append(acc)

    cp_tok.wait()
    for g in range(G):
        acc = accs[g]
        for j in range(NTOK):
            sl = pl.ds(j * RPW + g * L, L)
            ids = idx_tok_v[sl]
            av = val_tok_v[sl]
            tix = jnp.maximum(ids - TCUT, 0)
            tv = plsc.load_gather(aux_v, [zero_i, tix])
            acc = acc + jnp.where(ids >= TCUT, tv, av)
        accs[g] = acc

    cp_tab.wait()
    cp_tab2.wait()
    head = stab_v[pl.ds(0, L)]
    stab_v[pl.ds(0, L)] = jnp.where(lane == 0, jnp.zeros((L,), jnp.float32),
                                    head)

    for c in range(NCHUNK):
        cps[c % 2].wait()
        buf = ibufs[c % 2]

        def body(j, accs, buf=buf):
            out = []
            for g in range(G):
                ids = buf[pl.ds(j * RPW + g * L, L)]
                vs = plsc.load_gather(stab_v, [ids])
                out.append(accs[g] + vs)
            return tuple(out)

        accs = list(lax.fori_loop(0, CH, body, tuple(accs)))
        if c + 2 < NCHUNK:
            cps[c % 2] = pltpu.async_copy(
                seq_idx_hbm.at[pl.ds(sbase + (c + 2) * CH * RPW, CH * RPW)],
                ibufs[c % 2], sems[c % 2])

    for g in range(G):
        out_v[pl.ds(g * L, L)] = accs[g]
    pltpu.sync_copy(out_v, out_hbm.at[pl.ds(base, RPW)])


def kernel(float_fields, token_fields, token_seq, token_table, float_table,
           seq_table, bias):
    def permute(x):
        return x.reshape(NW, RPW, -1).transpose(0, 2, 1).reshape(-1)

    offsets = (jnp.arange(NTOK, dtype=jnp.int32) * VOCAB)[None, :]
    tok_idx = permute(token_fields.astype(jnp.int32) + offsets)
    tok_idx_cl = jnp.minimum(tok_idx, TCUT - 1)
    seq_idx = permute(token_seq.astype(jnp.int32))
    flt_t = permute(float_fields)

    ttab_a = lax.slice(token_table, (0, 0), (TCUT, 1)).reshape(-1)
    stab_a = lax.slice(seq_table, (0, 0), (SCUT, 1)).reshape(-1)
    aux = jnp.concatenate([
        lax.slice(token_table, (TCUT, 0), (TTOT, 1)).T,
        lax.slice(seq_table, (SCUT, 0), (VOCAB, 1)).T,
        float_table.T,
        bias[None, :],
        jnp.zeros((1, 128 - TTAIL - STAIL - NFLT - 1), jnp.float32)],
        axis=1)

    out = _fm_first_order(tok_idx_cl, tok_idx, seq_idx, flt_t,
                          ttab_a, stab_a, aux)
    return out.reshape(B, 1)
